# Initial kernel scaffold; baseline (speedup 1.0000x reference)
import jax
import jax.numpy as jnp
from jax.experimental import pallas as pl


def kernel(tokens, gate_w, gate_b, expert_w, expert_b):
    raise NotImplementedError("write your pallas kernel here")


# speedup vs baseline: 1.4773x; 1.4773x over previous
import functools

import jax
import jax.numpy as jnp
from jax.experimental import pallas as pl
from jax.experimental.pallas import tpu as pltpu

_B, _S, _D, _E = 2, 2048, 1024, 8


def _router_body(x_ref, gw_ref, gb_ref, w_ref):
    x = x_ref[0]
    logits = jax.lax.dot_general(
        x, gw_ref[...], (((1,), (1,)), ((), ())),
        preferred_element_type=jnp.float32)
    logits = logits + gb_ref[...]
    iota = jax.lax.broadcasted_iota(jnp.int32, (_S, _E), 1)
    m1 = jnp.max(logits, axis=1, keepdims=True)
    i1 = jnp.min(jnp.where(logits == m1, iota, _E), axis=1, keepdims=True)
    masked = jnp.where(iota == i1, -jnp.inf, logits)
    m2 = jnp.max(masked, axis=1, keepdims=True)
    i2 = jnp.min(jnp.where(masked == m2, iota, _E), axis=1, keepdims=True)

    def _smax_seq(c):
        mx = jnp.max(c, axis=0, keepdims=True)
        ex = jnp.exp(c - mx)
        return ex / jnp.sum(ex, axis=0, keepdims=True)

    w1 = _smax_seq(m1)
    w2 = _smax_seq(m2)
    w_ref[0] = (jnp.where(iota == i1, w1, 0.0)
                + jnp.where(iota == i2, w2, 0.0))


def _moe_body(w_ref, x_ref, we_ref, eb_ref, out_ref):
    e = pl.program_id(1)
    lane = jax.lax.broadcasted_iota(jnp.int32, w_ref.shape, 1)
    wcol = jnp.sum(jnp.where(lane == e, w_ref[...], 0.0), axis=1,
                   keepdims=True)
    xw = jax.lax.dot_general(
        x_ref[...], we_ref[0], (((1,), (1,)), ((), ())),
        preferred_element_type=jnp.float32)
    contrib = wcol * xw

    @pl.when(e == 0)
    def _():
        bias = jax.lax.dot_general(
            w_ref[...], eb_ref[...], (((1,), (0,)), ((), ())),
            preferred_element_type=jnp.float32)
        out_ref[...] = contrib + bias

    @pl.when(e != 0)
    def _():
        out_ref[...] += contrib


def kernel(tokens, gate_w, gate_b, expert_w, expert_b):
    w_dense = pl.pallas_call(
        _router_body,
        grid=(_B,),
        in_specs=[
            pl.BlockSpec((1, _S, _D), lambda b: (b, 0, 0)),
            pl.BlockSpec((_E, _D), lambda b: (0, 0)),
            pl.BlockSpec((1, _E), lambda b: (0, 0)),
        ],
        out_specs=pl.BlockSpec((1, _S, _E), lambda b: (b, 0, 0)),
        out_shape=jax.ShapeDtypeStruct((_B, _S, _E), jnp.float32),
    )(tokens, gate_w, gate_b.reshape(1, _E))

    n = _B * _S
    blk = 1024
    nb = n // blk
    x = tokens.reshape(n, _D).astype(jnp.bfloat16)
    wd = w_dense.reshape(n, _E)
    we = expert_w.astype(jnp.bfloat16)
    eb = expert_b.astype(jnp.float32)

    out = pl.pallas_call(
        _moe_body,
        grid=(nb, _E),
        in_specs=[
            pl.BlockSpec((blk, _E), lambda i, e: (i, 0)),
            pl.BlockSpec((blk, _D), lambda i, e: (i, 0)),
            pl.BlockSpec((1, _D, _D), lambda i, e: (e, 0, 0)),
            pl.BlockSpec((_E, _D), lambda i, e: (0, 0)),
        ],
        out_specs=pl.BlockSpec((blk, _D), lambda i, e: (i, 0)),
        out_shape=jax.ShapeDtypeStruct((n, _D), jnp.float32),
        compiler_params=pltpu.CompilerParams(
            dimension_semantics=("parallel", "arbitrary")),
    )(wd, x, we, eb)

    return out.reshape(_B, _S, _D)
